# baseline (speedup 1.0000x reference)
import functools

import jax
import jax.numpy as jnp
from jax import lax
from jax.experimental import pallas as pl
from jax.experimental.pallas import tpu as pltpu
from jax.experimental.pallas import tpu_sc as plsc

N = 10000
DIN = 128
DH = 16
DOUT = 128
E = 320000
MU = 0.1
P_EXP = 2.0
C0 = 2.0 * MU / P_EXP
EPS = 1e-8

NC = 2
NS = 16
NW = NC * NS
CH = 128
NCH = 80
EPW = NCH * CH
EP = NW * EPW
NP = 10240
RPT = NP // NS

_mesh = plsc.VectorSubcoreMesh(core_axis_name="c", subcore_axis_name="s")


def _deg_body(ei_hbm, ones_hbm, zero_hbm, out_hbm, ib, onesv, acc):
    cid = lax.axis_index("c")
    sid = lax.axis_index("s")
    wid = sid * NC + cid
    pltpu.sync_copy(zero_hbm, acc.at[pl.ds(sid * RPT, RPT)])
    pltpu.sync_copy(ones_hbm, onesv)
    plsc.subcore_barrier()

    def body(j, carry):
        pltpu.sync_copy(ei_hbm.at[wid, j], ib)
        pltpu.sync_copy(onesv, acc.at[ib.at[0]], add=True)
        return carry

    lax.fori_loop(0, NCH, body, 0)
    plsc.subcore_barrier()
    pltpu.sync_copy(acc.at[pl.ds(sid * RPT, RPT)],
                    out_hbm.at[pl.ds(cid * NP + sid * RPT, RPT)])


_deg_kernel = functools.partial(
    pl.kernel,
    out_type=jax.ShapeDtypeStruct((NC * NP, DOUT), jnp.float32),
    mesh=_mesh,
    scratch_types=[
        pltpu.VMEM((2, CH), jnp.int32),
        pltpu.VMEM((CH, DOUT), jnp.float32),
        pltpu.VMEM_SHARED((NP, DOUT), jnp.float32),
    ],
)(_deg_body)


def _spmm_body(g_hbm, ei_hbm, zero_hbm, out_hbm,
               ib0, ib1, gv0, gv1, acc, sem0, sem1):
    cid = lax.axis_index("c")
    sid = lax.axis_index("s")
    wid = sid * NC + cid
    pltpu.sync_copy(zero_hbm, acc.at[pl.ds(sid * RPT, RPT)])
    pltpu.sync_copy(ei_hbm.at[wid, 0], ib0)
    plsc.subcore_barrier()
    pltpu.async_copy(g_hbm.at[ib0.at[1]], gv0, sem0)
    pltpu.sync_copy(ei_hbm.at[wid, 1], ib1)

    def body(j2, carry):
        a = 2 * j2
        b = a + 1
        pltpu.async_copy(g_hbm.at[ib1.at[1]], gv1, sem1)
        pltpu.make_async_copy(g_hbm.at[ib0.at[1]], gv0, sem0).wait()
        pltpu.sync_copy(gv0, acc.at[ib0.at[0]], add=True)
        na = jnp.minimum(a + 2, NCH - 1)
        pltpu.sync_copy(ei_hbm.at[wid, na], ib0)
        pltpu.async_copy(g_hbm.at[ib0.at[1]], gv0, sem0)
        pltpu.make_async_copy(g_hbm.at[ib1.at[1]], gv1, sem1).wait()
        pltpu.sync_copy(gv1, acc.at[ib1.at[0]], add=True)
        nb = jnp.minimum(b + 2, NCH - 1)
        pltpu.sync_copy(ei_hbm.at[wid, nb], ib1)
        return carry

    lax.fori_loop(0, NCH // 2, body, 0)
    pltpu.make_async_copy(g_hbm.at[ib0.at[1]], gv0, sem0).wait()
    plsc.subcore_barrier()
    pltpu.sync_copy(acc.at[pl.ds(sid * RPT, RPT)],
                    out_hbm.at[pl.ds(cid * NP + sid * RPT, RPT)])


_spmm_kernel = functools.partial(
    pl.kernel,
    out_type=jax.ShapeDtypeStruct((NC * NP, DOUT), jnp.float32),
    mesh=_mesh,
    scratch_types=[
        pltpu.VMEM((2, CH), jnp.int32),
        pltpu.VMEM((2, CH), jnp.int32),
        pltpu.VMEM((CH, DOUT), jnp.float32),
        pltpu.VMEM((CH, DOUT), jnp.float32),
        pltpu.VMEM_SHARED((NP, DOUT), jnp.float32),
        pltpu.SemaphoreType.DMA,
        pltpu.SemaphoreType.DMA,
    ],
)(_spmm_body)


_BR = 1000


def _dense_body(x_ref, w1_ref, b1_ref, w2_ref, b2_ref, h_ref):
    hm = jnp.dot(x_ref[...], w1_ref[...], preferred_element_type=jnp.float32)
    hm = jnp.maximum(hm + b1_ref[...], 0.0)
    h_ref[...] = (jnp.dot(hm, w2_ref[...], preferred_element_type=jnp.float32)
                  + b2_ref[...])


def _dense(x, w1, b1, w2, b2):
    return pl.pallas_call(
        _dense_body,
        grid=(N // _BR,),
        in_specs=[
            pl.BlockSpec((_BR, DIN), lambda i: (i, 0)),
            pl.BlockSpec((DIN, DH), lambda i: (0, 0)),
            pl.BlockSpec((1, DH), lambda i: (0, 0)),
            pl.BlockSpec((DH, DOUT), lambda i: (0, 0)),
            pl.BlockSpec((1, DOUT), lambda i: (0, 0)),
        ],
        out_specs=pl.BlockSpec((_BR, DOUT), lambda i: (i, 0)),
        out_shape=jax.ShapeDtypeStruct((N, DOUT), jnp.float32),
    )(x, w1, b1, w2, b2)


def _factors_body(d0_ref, d1_ref, h_ref, g1_ref, ad_ref, b8_ref, di_ref):
    dega = d0_ref[:, 0:1] + d1_ref[:, 0:1]
    deg = jnp.maximum(dega, EPS)
    dinv = deg ** -0.5
    msum = dega / deg
    alpha = 1.0 / (msum + C0)
    beta = C0 * alpha
    g1_ref[...] = dinv * h_ref[...]
    ad_ref[...] = jnp.broadcast_to(alpha * dinv, (_BR, 8))
    b8_ref[...] = jnp.broadcast_to(beta, (_BR, 8))
    di_ref[...] = jnp.broadcast_to(dinv, (_BR, 8))


def _factors(d0, d1, h):
    return pl.pallas_call(
        _factors_body,
        grid=(N // _BR,),
        in_specs=[
            pl.BlockSpec((_BR, DOUT), lambda i: (i, 0)),
            pl.BlockSpec((_BR, DOUT), lambda i: (i, 0)),
            pl.BlockSpec((_BR, DOUT), lambda i: (i, 0)),
        ],
        out_specs=[
            pl.BlockSpec((_BR, DOUT), lambda i: (i, 0)),
            pl.BlockSpec((_BR, 8), lambda i: (i, 0)),
            pl.BlockSpec((_BR, 8), lambda i: (i, 0)),
            pl.BlockSpec((_BR, 8), lambda i: (i, 0)),
        ],
        out_shape=[
            jax.ShapeDtypeStruct((N, DOUT), jnp.float32),
            jax.ShapeDtypeStruct((N, 8), jnp.float32),
            jax.ShapeDtypeStruct((N, 8), jnp.float32),
            jax.ShapeDtypeStruct((N, 8), jnp.float32),
        ],
    )(d0, d1, h)


def _mid_body(s0_ref, s1_ref, h_ref, ad_ref, b8_ref, di_ref, g2_ref):
    f = (ad_ref[:, 0:1] * (s0_ref[...] + s1_ref[...])
         + b8_ref[:, 0:1] * h_ref[...])
    g2_ref[...] = di_ref[:, 0:1] * f


def _mid(s0, s1, h, ad, b8, di):
    return pl.pallas_call(
        _mid_body,
        grid=(N // _BR,),
        in_specs=[
            pl.BlockSpec((_BR, DOUT), lambda i: (i, 0)),
            pl.BlockSpec((_BR, DOUT), lambda i: (i, 0)),
            pl.BlockSpec((_BR, DOUT), lambda i: (i, 0)),
            pl.BlockSpec((_BR, 8), lambda i: (i, 0)),
            pl.BlockSpec((_BR, 8), lambda i: (i, 0)),
            pl.BlockSpec((_BR, 8), lambda i: (i, 0)),
        ],
        out_specs=pl.BlockSpec((_BR, DOUT), lambda i: (i, 0)),
        out_shape=jax.ShapeDtypeStruct((N, DOUT), jnp.float32),
    )(s0, s1, h, ad, b8, di)


def _final_body(s0_ref, s1_ref, h_ref, ad_ref, b8_ref, o_ref):
    f = (ad_ref[:, 0:1] * (s0_ref[...] + s1_ref[...])
         + b8_ref[:, 0:1] * h_ref[...])
    m = jnp.max(f, axis=1, keepdims=True)
    sh = f - m
    o_ref[...] = sh - jnp.log(jnp.sum(jnp.exp(sh), axis=1, keepdims=True))


def _final(s0, s1, h, ad, b8):
    return pl.pallas_call(
        _final_body,
        grid=(N // _BR,),
        in_specs=[
            pl.BlockSpec((_BR, DOUT), lambda i: (i, 0)),
            pl.BlockSpec((_BR, DOUT), lambda i: (i, 0)),
            pl.BlockSpec((_BR, DOUT), lambda i: (i, 0)),
            pl.BlockSpec((_BR, 8), lambda i: (i, 0)),
            pl.BlockSpec((_BR, 8), lambda i: (i, 0)),
        ],
        out_specs=pl.BlockSpec((_BR, DOUT), lambda i: (i, 0)),
        out_shape=jax.ShapeDtypeStruct((N, DOUT), jnp.float32),
    )(s0, s1, h, ad, b8)


@jax.jit
def kernel(x, edge_index, W_lin1, b_lin1, W_conv, b_conv):
    pad = EP - E
    row = jnp.concatenate(
        [edge_index[0].astype(jnp.int32), jnp.full((pad,), N, jnp.int32)])
    col = jnp.concatenate(
        [edge_index[1].astype(jnp.int32), jnp.zeros((pad,), jnp.int32)])
    ei2 = jnp.stack(
        [row.reshape(NW, NCH, CH), col.reshape(NW, NCH, CH)], axis=2)
    ones1 = jnp.ones((CH, DOUT), jnp.float32)
    zeroD = jnp.zeros((RPT, DOUT), jnp.float32)

    h = _dense(x, W_lin1, b_lin1.reshape(1, DH), W_conv, b_conv.reshape(1, DOUT))
    degp = _deg_kernel(ei2, ones1, zeroD)
    g1, ad, b8, di = _factors(degp[:N], degp[NP:NP + N], h)
    sp = _spmm_kernel(g1, ei2, zeroD)
    g2 = _mid(sp[:N], sp[NP:NP + N], h, ad, b8, di)
    sp2 = _spmm_kernel(g2, ei2, zeroD)
    return _final(sp2[:N], sp2[NP:NP + N], h, ad, b8)


# speedup vs baseline: 1.0380x; 1.0380x over previous
import functools

import jax
import jax.numpy as jnp
from jax import lax
from jax.experimental import pallas as pl
from jax.experimental.pallas import tpu as pltpu
from jax.experimental.pallas import tpu_sc as plsc

N = 10000
DIN = 128
DH = 16
DOUT = 128
E = 320000
MU = 0.1
P_EXP = 2.0
C0 = 2.0 * MU / P_EXP
EPS = 1e-8

NC = 2
NS = 16
NW = NC * NS
CH = 128
NCH = 80
EPW = NCH * CH
EP = NW * EPW
NP = 10240
RPT = NP // NS

_mesh = plsc.VectorSubcoreMesh(core_axis_name="c", subcore_axis_name="s")


def _deg_body(ei_hbm, ones_hbm, zero_hbm, out_hbm, ib0, ib1, onesv, acc,
              si0, si1):
    cid = lax.axis_index("c")
    sid = lax.axis_index("s")
    wid = sid * NC + cid
    pltpu.sync_copy(zero_hbm, acc.at[pl.ds(sid * RPT, RPT)])
    pltpu.sync_copy(ones_hbm, onesv)
    pltpu.async_copy(ei_hbm.at[wid, 0], ib0, si0)
    pltpu.async_copy(ei_hbm.at[wid, 1], ib1, si1)
    plsc.subcore_barrier()

    def body(j2, carry):
        a = 2 * j2
        b = a + 1
        pltpu.make_async_copy(ei_hbm.at[wid, a], ib0, si0).wait()
        pltpu.sync_copy(onesv, acc.at[ib0.at[0]], add=True)
        pltpu.async_copy(ei_hbm.at[wid, jnp.minimum(a + 2, NCH - 1)], ib0, si0)
        pltpu.make_async_copy(ei_hbm.at[wid, b], ib1, si1).wait()
        pltpu.sync_copy(onesv, acc.at[ib1.at[0]], add=True)
        pltpu.async_copy(ei_hbm.at[wid, jnp.minimum(b + 2, NCH - 1)], ib1, si1)
        return carry

    lax.fori_loop(0, NCH // 2, body, 0)
    pltpu.make_async_copy(ei_hbm.at[wid, 0], ib0, si0).wait()
    pltpu.make_async_copy(ei_hbm.at[wid, 0], ib1, si1).wait()
    plsc.subcore_barrier()
    pltpu.sync_copy(acc.at[pl.ds(sid * RPT, RPT)],
                    out_hbm.at[pl.ds(cid * NP + sid * RPT, RPT)])


_deg_kernel = functools.partial(
    pl.kernel,
    out_type=jax.ShapeDtypeStruct((NC * NP, DOUT), jnp.float32),
    mesh=_mesh,
    scratch_types=[
        pltpu.VMEM((2, CH), jnp.int32),
        pltpu.VMEM((2, CH), jnp.int32),
        pltpu.VMEM((CH, DOUT), jnp.float32),
        pltpu.VMEM_SHARED((NP, DOUT), jnp.float32),
        pltpu.SemaphoreType.DMA,
        pltpu.SemaphoreType.DMA,
    ],
)(_deg_body)


def _spmm_body(g_hbm, ei_hbm, zero_hbm, out_hbm,
               ib0, ib1, ib2, ib3, gv0, gv1, acc,
               si0, si1, si2, si3, sg0, sg1):
    cid = lax.axis_index("c")
    sid = lax.axis_index("s")
    wid = sid * NC + cid
    pltpu.sync_copy(zero_hbm, acc.at[pl.ds(sid * RPT, RPT)])
    pltpu.async_copy(ei_hbm.at[wid, 0], ib0, si0)
    pltpu.async_copy(ei_hbm.at[wid, 1], ib1, si1)
    pltpu.async_copy(ei_hbm.at[wid, 2], ib2, si2)
    pltpu.async_copy(ei_hbm.at[wid, 3], ib3, si3)
    plsc.subcore_barrier()
    pltpu.make_async_copy(ei_hbm.at[wid, 0], ib0, si0).wait()
    pltpu.async_copy(g_hbm.at[ib0.at[1]], gv0, sg0)
    pltpu.make_async_copy(ei_hbm.at[wid, 1], ib1, si1).wait()

    def body(j4, carry):
        c0 = 4 * j4
        pltpu.async_copy(g_hbm.at[ib1.at[1]], gv1, sg1)
        pltpu.make_async_copy(g_hbm.at[ib0.at[1]], gv0, sg0).wait()
        pltpu.sync_copy(gv0, acc.at[ib0.at[0]], add=True)
        pltpu.async_copy(ei_hbm.at[wid, jnp.minimum(c0 + 4, NCH - 1)], ib0, si0)
        pltpu.make_async_copy(ei_hbm.at[wid, 0], ib2, si2).wait()
        pltpu.async_copy(g_hbm.at[ib2.at[1]], gv0, sg0)
        pltpu.make_async_copy(g_hbm.at[ib1.at[1]], gv1, sg1).wait()
        pltpu.sync_copy(gv1, acc.at[ib1.at[0]], add=True)
        pltpu.async_copy(ei_hbm.at[wid, jnp.minimum(c0 + 5, NCH - 1)], ib1, si1)
        pltpu.make_async_copy(ei_hbm.at[wid, 0], ib3, si3).wait()
        pltpu.async_copy(g_hbm.at[ib3.at[1]], gv1, sg1)
        pltpu.make_async_copy(g_hbm.at[ib2.at[1]], gv0, sg0).wait()
        pltpu.sync_copy(gv0, acc.at[ib2.at[0]], add=True)
        pltpu.async_copy(ei_hbm.at[wid, jnp.minimum(c0 + 6, NCH - 1)], ib2, si2)
        pltpu.make_async_copy(ei_hbm.at[wid, 0], ib0, si0).wait()
        pltpu.async_copy(g_hbm.at[ib0.at[1]], gv0, sg0)
        pltpu.make_async_copy(g_hbm.at[ib3.at[1]], gv1, sg1).wait()
        pltpu.sync_copy(gv1, acc.at[ib3.at[0]], add=True)
        pltpu.async_copy(ei_hbm.at[wid, jnp.minimum(c0 + 7, NCH - 1)], ib3, si3)
        pltpu.make_async_copy(ei_hbm.at[wid, 0], ib1, si1).wait()
        return carry

    lax.fori_loop(0, NCH // 4, body, 0)
    pltpu.make_async_copy(g_hbm.at[ib0.at[1]], gv0, sg0).wait()
    pltpu.make_async_copy(ei_hbm.at[wid, 0], ib2, si2).wait()
    pltpu.make_async_copy(ei_hbm.at[wid, 0], ib3, si3).wait()
    plsc.subcore_barrier()
    pltpu.sync_copy(acc.at[pl.ds(sid * RPT, RPT)],
                    out_hbm.at[pl.ds(cid * NP + sid * RPT, RPT)])


_spmm_kernel = functools.partial(
    pl.kernel,
    out_type=jax.ShapeDtypeStruct((NC * NP, DOUT), jnp.float32),
    mesh=_mesh,
    scratch_types=[
        pltpu.VMEM((2, CH), jnp.int32),
        pltpu.VMEM((2, CH), jnp.int32),
        pltpu.VMEM((2, CH), jnp.int32),
        pltpu.VMEM((2, CH), jnp.int32),
        pltpu.VMEM((CH, DOUT), jnp.float32),
        pltpu.VMEM((CH, DOUT), jnp.float32),
        pltpu.VMEM_SHARED((NP, DOUT), jnp.float32),
        pltpu.SemaphoreType.DMA,
        pltpu.SemaphoreType.DMA,
        pltpu.SemaphoreType.DMA,
        pltpu.SemaphoreType.DMA,
        pltpu.SemaphoreType.DMA,
        pltpu.SemaphoreType.DMA,
    ],
)(_spmm_body)


_BR = 1000


def _dense_body(x_ref, w1_ref, b1_ref, w2_ref, b2_ref, h_ref):
    hm = jnp.dot(x_ref[...], w1_ref[...], preferred_element_type=jnp.float32)
    hm = jnp.maximum(hm + b1_ref[...], 0.0)
    h_ref[...] = (jnp.dot(hm, w2_ref[...], preferred_element_type=jnp.float32)
                  + b2_ref[...])


def _dense(x, w1, b1, w2, b2):
    return pl.pallas_call(
        _dense_body,
        grid=(N // _BR,),
        in_specs=[
            pl.BlockSpec((_BR, DIN), lambda i: (i, 0)),
            pl.BlockSpec((DIN, DH), lambda i: (0, 0)),
            pl.BlockSpec((1, DH), lambda i: (0, 0)),
            pl.BlockSpec((DH, DOUT), lambda i: (0, 0)),
            pl.BlockSpec((1, DOUT), lambda i: (0, 0)),
        ],
        out_specs=pl.BlockSpec((_BR, DOUT), lambda i: (i, 0)),
        out_shape=jax.ShapeDtypeStruct((N, DOUT), jnp.float32),
    )(x, w1, b1, w2, b2)


def _factors_body(d0_ref, d1_ref, h_ref, g1_ref, ad_ref, b8_ref, di_ref):
    dega = d0_ref[:, 0:1] + d1_ref[:, 0:1]
    deg = jnp.maximum(dega, EPS)
    dinv = deg ** -0.5
    msum = dega / deg
    alpha = 1.0 / (msum + C0)
    beta = C0 * alpha
    g1_ref[...] = dinv * h_ref[...]
    ad_ref[...] = jnp.broadcast_to(alpha * dinv, (_BR, 8))
    b8_ref[...] = jnp.broadcast_to(beta, (_BR, 8))
    di_ref[...] = jnp.broadcast_to(dinv, (_BR, 8))


def _factors(d0, d1, h):
    return pl.pallas_call(
        _factors_body,
        grid=(N // _BR,),
        in_specs=[
            pl.BlockSpec((_BR, DOUT), lambda i: (i, 0)),
            pl.BlockSpec((_BR, DOUT), lambda i: (i, 0)),
            pl.BlockSpec((_BR, DOUT), lambda i: (i, 0)),
        ],
        out_specs=[
            pl.BlockSpec((_BR, DOUT), lambda i: (i, 0)),
            pl.BlockSpec((_BR, 8), lambda i: (i, 0)),
            pl.BlockSpec((_BR, 8), lambda i: (i, 0)),
            pl.BlockSpec((_BR, 8), lambda i: (i, 0)),
        ],
        out_shape=[
            jax.ShapeDtypeStruct((N, DOUT), jnp.float32),
            jax.ShapeDtypeStruct((N, 8), jnp.float32),
            jax.ShapeDtypeStruct((N, 8), jnp.float32),
            jax.ShapeDtypeStruct((N, 8), jnp.float32),
        ],
    )(d0, d1, h)


def _mid_body(s0_ref, s1_ref, h_ref, ad_ref, b8_ref, di_ref, g2_ref):
    f = (ad_ref[:, 0:1] * (s0_ref[...] + s1_ref[...])
         + b8_ref[:, 0:1] * h_ref[...])
    g2_ref[...] = di_ref[:, 0:1] * f


def _mid(s0, s1, h, ad, b8, di):
    return pl.pallas_call(
        _mid_body,
        grid=(N // _BR,),
        in_specs=[
            pl.BlockSpec((_BR, DOUT), lambda i: (i, 0)),
            pl.BlockSpec((_BR, DOUT), lambda i: (i, 0)),
            pl.BlockSpec((_BR, DOUT), lambda i: (i, 0)),
            pl.BlockSpec((_BR, 8), lambda i: (i, 0)),
            pl.BlockSpec((_BR, 8), lambda i: (i, 0)),
            pl.BlockSpec((_BR, 8), lambda i: (i, 0)),
        ],
        out_specs=pl.BlockSpec((_BR, DOUT), lambda i: (i, 0)),
        out_shape=jax.ShapeDtypeStruct((N, DOUT), jnp.float32),
    )(s0, s1, h, ad, b8, di)


def _final_body(s0_ref, s1_ref, h_ref, ad_ref, b8_ref, o_ref):
    f = (ad_ref[:, 0:1] * (s0_ref[...] + s1_ref[...])
         + b8_ref[:, 0:1] * h_ref[...])
    m = jnp.max(f, axis=1, keepdims=True)
    sh = f - m
    o_ref[...] = sh - jnp.log(jnp.sum(jnp.exp(sh), axis=1, keepdims=True))


def _final(s0, s1, h, ad, b8):
    return pl.pallas_call(
        _final_body,
        grid=(N // _BR,),
        in_specs=[
            pl.BlockSpec((_BR, DOUT), lambda i: (i, 0)),
            pl.BlockSpec((_BR, DOUT), lambda i: (i, 0)),
            pl.BlockSpec((_BR, DOUT), lambda i: (i, 0)),
            pl.BlockSpec((_BR, 8), lambda i: (i, 0)),
            pl.BlockSpec((_BR, 8), lambda i: (i, 0)),
        ],
        out_specs=pl.BlockSpec((_BR, DOUT), lambda i: (i, 0)),
        out_shape=jax.ShapeDtypeStruct((N, DOUT), jnp.float32),
    )(s0, s1, h, ad, b8)


@jax.jit
def kernel(x, edge_index, W_lin1, b_lin1, W_conv, b_conv):
    pad = EP - E
    row = jnp.concatenate(
        [edge_index[0].astype(jnp.int32), jnp.full((pad,), N, jnp.int32)])
    col = jnp.concatenate(
        [edge_index[1].astype(jnp.int32), jnp.zeros((pad,), jnp.int32)])
    ei2 = jnp.stack(
        [row.reshape(NW, NCH, CH), col.reshape(NW, NCH, CH)], axis=2)
    ones1 = jnp.ones((CH, DOUT), jnp.float32)
    zeroD = jnp.zeros((RPT, DOUT), jnp.float32)

    h = _dense(x, W_lin1, b_lin1.reshape(1, DH), W_conv, b_conv.reshape(1, DOUT))
    degp = _deg_kernel(ei2, ones1, zeroD)
    g1, ad, b8, di = _factors(degp[:N], degp[NP:NP + N], h)
    sp = _spmm_kernel(g1, ei2, zeroD)
    g2 = _mid(sp[:N], sp[NP:NP + N], h, ad, b8, di)
    sp2 = _spmm_kernel(g2, ei2, zeroD)
    return _final(sp2[:N], sp2[NP:NP + N], h, ad, b8)


# speedup vs baseline: 2.8652x; 2.7603x over previous
import functools

import jax
import jax.numpy as jnp
from jax import lax
from jax.experimental import pallas as pl
from jax.experimental.pallas import tpu as pltpu
from jax.experimental.pallas import tpu_sc as plsc

N = 10000
DIN = 128
DH = 16
DOUT = 128
E = 320000
MU = 0.1
P_EXP = 2.0
C0 = 2.0 * MU / P_EXP
EPS = 1e-8

NC = 2
NS = 16
NW = NC * NS
CH = 128
NCH = 80
EPW = NCH * CH
EP = NW * EPW
NP = 10240
RPT = NP // NS

_mesh = plsc.VectorSubcoreMesh(core_axis_name="c", subcore_axis_name="s")


def _deg_body(ei_hbm, ones_hbm, zero_hbm, out_hbm, ib0, ib1, onesv, acc,
              si0, si1):
    cid = lax.axis_index("c")
    sid = lax.axis_index("s")
    wid = sid * NC + cid
    pltpu.sync_copy(zero_hbm, acc.at[pl.ds(sid * RPT, RPT)])
    pltpu.sync_copy(ones_hbm, onesv)
    pltpu.async_copy(ei_hbm.at[wid, 0], ib0, si0)
    pltpu.async_copy(ei_hbm.at[wid, 1], ib1, si1)
    plsc.subcore_barrier()

    def body(j2, carry):
        a = 2 * j2
        b = a + 1
        pltpu.make_async_copy(ei_hbm.at[wid, a], ib0, si0).wait()
        pltpu.sync_copy(onesv, acc.at[ib0.at[0]], add=True)
        pltpu.async_copy(ei_hbm.at[wid, jnp.minimum(a + 2, NCH - 1)], ib0, si0)
        pltpu.make_async_copy(ei_hbm.at[wid, b], ib1, si1).wait()
        pltpu.sync_copy(onesv, acc.at[ib1.at[0]], add=True)
        pltpu.async_copy(ei_hbm.at[wid, jnp.minimum(b + 2, NCH - 1)], ib1, si1)
        return carry

    lax.fori_loop(0, NCH // 2, body, 0)
    pltpu.make_async_copy(ei_hbm.at[wid, 0], ib0, si0).wait()
    pltpu.make_async_copy(ei_hbm.at[wid, 0], ib1, si1).wait()
    plsc.subcore_barrier()
    pltpu.sync_copy(acc.at[pl.ds(sid * RPT, RPT)],
                    out_hbm.at[pl.ds(cid * NP + sid * RPT, RPT)])


_deg_kernel = functools.partial(
    pl.kernel,
    out_type=jax.ShapeDtypeStruct((NC * NP, DOUT), jnp.float32),
    mesh=_mesh,
    scratch_types=[
        pltpu.VMEM((2, CH), jnp.int32),
        pltpu.VMEM((2, CH), jnp.int32),
        pltpu.VMEM((CH, DOUT), jnp.float32),
        pltpu.VMEM_SHARED((NP, DOUT), jnp.float32),
        pltpu.SemaphoreType.DMA,
        pltpu.SemaphoreType.DMA,
    ],
)(_deg_body)


def _spmm_body(g_hbm, ei_hbm, zero_hbm, out_hbm,
               ib0, ib1, ib2, ib3, gv0, gv1, acc,
               si0, si1, si2, si3, sg0, sg1):
    cid = lax.axis_index("c")
    sid = lax.axis_index("s")
    wid = sid * NC + cid
    pltpu.sync_copy(zero_hbm, acc.at[pl.ds(sid * RPT, RPT)])
    pltpu.async_copy(ei_hbm.at[wid, 0], ib0, si0)
    pltpu.async_copy(ei_hbm.at[wid, 1], ib1, si1)
    pltpu.async_copy(ei_hbm.at[wid, 2], ib2, si2)
    pltpu.async_copy(ei_hbm.at[wid, 3], ib3, si3)
    plsc.subcore_barrier()
    pltpu.make_async_copy(ei_hbm.at[wid, 0], ib0, si0).wait()
    pltpu.async_copy(g_hbm.at[ib0.at[1]], gv0, sg0)
    pltpu.make_async_copy(ei_hbm.at[wid, 1], ib1, si1).wait()

    def body(j4, carry):
        c0 = 4 * j4
        pltpu.async_copy(g_hbm.at[ib1.at[1]], gv1, sg1)
        pltpu.make_async_copy(g_hbm.at[ib0.at[1]], gv0, sg0).wait()
        pltpu.sync_copy(gv0, acc.at[ib0.at[0]], add=True)
        pltpu.async_copy(ei_hbm.at[wid, jnp.minimum(c0 + 4, NCH - 1)], ib0, si0)
        pltpu.make_async_copy(ei_hbm.at[wid, 0], ib2, si2).wait()
        pltpu.async_copy(g_hbm.at[ib2.at[1]], gv0, sg0)
        pltpu.make_async_copy(g_hbm.at[ib1.at[1]], gv1, sg1).wait()
        pltpu.sync_copy(gv1, acc.at[ib1.at[0]], add=True)
        pltpu.async_copy(ei_hbm.at[wid, jnp.minimum(c0 + 5, NCH - 1)], ib1, si1)
        pltpu.make_async_copy(ei_hbm.at[wid, 0], ib3, si3).wait()
        pltpu.async_copy(g_hbm.at[ib3.at[1]], gv1, sg1)
        pltpu.make_async_copy(g_hbm.at[ib2.at[1]], gv0, sg0).wait()
        pltpu.sync_copy(gv0, acc.at[ib2.at[0]], add=True)
        pltpu.async_copy(ei_hbm.at[wid, jnp.minimum(c0 + 6, NCH - 1)], ib2, si2)
        pltpu.make_async_copy(ei_hbm.at[wid, 0], ib0, si0).wait()
        pltpu.async_copy(g_hbm.at[ib0.at[1]], gv0, sg0)
        pltpu.make_async_copy(g_hbm.at[ib3.at[1]], gv1, sg1).wait()
        pltpu.sync_copy(gv1, acc.at[ib3.at[0]], add=True)
        pltpu.async_copy(ei_hbm.at[wid, jnp.minimum(c0 + 7, NCH - 1)], ib3, si3)
        pltpu.make_async_copy(ei_hbm.at[wid, 0], ib1, si1).wait()
        return carry

    lax.fori_loop(0, NCH // 4, body, 0)
    pltpu.make_async_copy(g_hbm.at[ib0.at[1]], gv0, sg0).wait()
    pltpu.make_async_copy(ei_hbm.at[wid, 0], ib2, si2).wait()
    pltpu.make_async_copy(ei_hbm.at[wid, 0], ib3, si3).wait()
    plsc.subcore_barrier()
    pltpu.sync_copy(acc.at[pl.ds(sid * RPT, RPT)],
                    out_hbm.at[pl.ds(cid * NP + sid * RPT, RPT)])


_spmm_kernel = functools.partial(
    pl.kernel,
    out_type=jax.ShapeDtypeStruct((NC * NP, DOUT), jnp.float32),
    mesh=_mesh,
    scratch_types=[
        pltpu.VMEM((2, CH), jnp.int32),
        pltpu.VMEM((2, CH), jnp.int32),
        pltpu.VMEM((2, CH), jnp.int32),
        pltpu.VMEM((2, CH), jnp.int32),
        pltpu.VMEM((CH, DOUT), jnp.float32),
        pltpu.VMEM((CH, DOUT), jnp.float32),
        pltpu.VMEM_SHARED((NP, DOUT), jnp.float32),
        pltpu.SemaphoreType.DMA,
        pltpu.SemaphoreType.DMA,
        pltpu.SemaphoreType.DMA,
        pltpu.SemaphoreType.DMA,
        pltpu.SemaphoreType.DMA,
        pltpu.SemaphoreType.DMA,
    ],
)(_spmm_body)


_BR = 1000


def _dense_body(x_ref, w1_ref, b1_ref, w2_ref, b2_ref, h_ref):
    hm = jnp.dot(x_ref[...], w1_ref[...], preferred_element_type=jnp.float32)
    hm = jnp.maximum(hm + b1_ref[...], 0.0)
    h_ref[...] = (jnp.dot(hm, w2_ref[...], preferred_element_type=jnp.float32)
                  + b2_ref[...])


def _dense(x, w1, b1, w2, b2):
    return pl.pallas_call(
        _dense_body,
        grid=(N // _BR,),
        in_specs=[
            pl.BlockSpec((_BR, DIN), lambda i: (i, 0)),
            pl.BlockSpec((DIN, DH), lambda i: (0, 0)),
            pl.BlockSpec((1, DH), lambda i: (0, 0)),
            pl.BlockSpec((DH, DOUT), lambda i: (0, 0)),
            pl.BlockSpec((1, DOUT), lambda i: (0, 0)),
        ],
        out_specs=pl.BlockSpec((_BR, DOUT), lambda i: (i, 0)),
        out_shape=jax.ShapeDtypeStruct((N, DOUT), jnp.float32),
    )(x, w1, b1, w2, b2)


def _factors_body(d0_ref, d1_ref, h_ref, g1_ref, ad_ref, b8_ref, di_ref):
    dega = d0_ref[:, 0:1] + d1_ref[:, 0:1]
    deg = jnp.maximum(dega, EPS)
    dinv = deg ** -0.5
    msum = dega / deg
    alpha = 1.0 / (msum + C0)
    beta = C0 * alpha
    g1_ref[...] = dinv * h_ref[...]
    ad_ref[...] = jnp.broadcast_to(alpha * dinv, (_BR, 8))
    b8_ref[...] = jnp.broadcast_to(beta, (_BR, 8))
    di_ref[...] = jnp.broadcast_to(dinv, (_BR, 8))


def _factors(d0, d1, h):
    return pl.pallas_call(
        _factors_body,
        grid=(N // _BR,),
        in_specs=[
            pl.BlockSpec((_BR, DOUT), lambda i: (i, 0)),
            pl.BlockSpec((_BR, DOUT), lambda i: (i, 0)),
            pl.BlockSpec((_BR, DOUT), lambda i: (i, 0)),
        ],
        out_specs=[
            pl.BlockSpec((_BR, DOUT), lambda i: (i, 0)),
            pl.BlockSpec((_BR, 8), lambda i: (i, 0)),
            pl.BlockSpec((_BR, 8), lambda i: (i, 0)),
            pl.BlockSpec((_BR, 8), lambda i: (i, 0)),
        ],
        out_shape=[
            jax.ShapeDtypeStruct((N, DOUT), jnp.float32),
            jax.ShapeDtypeStruct((N, 8), jnp.float32),
            jax.ShapeDtypeStruct((N, 8), jnp.float32),
            jax.ShapeDtypeStruct((N, 8), jnp.float32),
        ],
    )(d0, d1, h)


def _mid_body(s0_ref, s1_ref, h_ref, ad_ref, b8_ref, di_ref, g2_ref):
    f = (ad_ref[:, 0:1] * (s0_ref[...] + s1_ref[...])
         + b8_ref[:, 0:1] * h_ref[...])
    g2_ref[...] = di_ref[:, 0:1] * f


def _mid(s0, s1, h, ad, b8, di):
    return pl.pallas_call(
        _mid_body,
        grid=(N // _BR,),
        in_specs=[
            pl.BlockSpec((_BR, DOUT), lambda i: (i, 0)),
            pl.BlockSpec((_BR, DOUT), lambda i: (i, 0)),
            pl.BlockSpec((_BR, DOUT), lambda i: (i, 0)),
            pl.BlockSpec((_BR, 8), lambda i: (i, 0)),
            pl.BlockSpec((_BR, 8), lambda i: (i, 0)),
            pl.BlockSpec((_BR, 8), lambda i: (i, 0)),
        ],
        out_specs=pl.BlockSpec((_BR, DOUT), lambda i: (i, 0)),
        out_shape=jax.ShapeDtypeStruct((N, DOUT), jnp.float32),
    )(s0, s1, h, ad, b8, di)


def _final_body(s0_ref, s1_ref, h_ref, ad_ref, b8_ref, o_ref):
    f = (ad_ref[:, 0:1] * (s0_ref[...] + s1_ref[...])
         + b8_ref[:, 0:1] * h_ref[...])
    m = jnp.max(f, axis=1, keepdims=True)
    sh = f - m
    o_ref[...] = sh - jnp.log(jnp.sum(jnp.exp(sh), axis=1, keepdims=True))


def _final(s0, s1, h, ad, b8):
    return pl.pallas_call(
        _final_body,
        grid=(N // _BR,),
        in_specs=[
            pl.BlockSpec((_BR, DOUT), lambda i: (i, 0)),
            pl.BlockSpec((_BR, DOUT), lambda i: (i, 0)),
            pl.BlockSpec((_BR, DOUT), lambda i: (i, 0)),
            pl.BlockSpec((_BR, 8), lambda i: (i, 0)),
            pl.BlockSpec((_BR, 8), lambda i: (i, 0)),
        ],
        out_specs=pl.BlockSpec((_BR, DOUT), lambda i: (i, 0)),
        out_shape=jax.ShapeDtypeStruct((N, DOUT), jnp.float32),
    )(s0, s1, h, ad, b8)


@jax.jit
def kernel(x, edge_index, W_lin1, b_lin1, W_conv, b_conv):
    pad = EP - E
    padr = N + jnp.arange(pad, dtype=jnp.int32) % (NP - N)
    padc = jnp.arange(pad, dtype=jnp.int32) % N
    row = jnp.concatenate([edge_index[0].astype(jnp.int32), padr])
    col = jnp.concatenate([edge_index[1].astype(jnp.int32), padc])
    ei2 = jnp.stack(
        [row.reshape(NW, NCH, CH), col.reshape(NW, NCH, CH)], axis=2)
    ones1 = jnp.ones((CH, DOUT), jnp.float32)
    zeroD = jnp.zeros((RPT, DOUT), jnp.float32)

    h = _dense(x, W_lin1, b_lin1.reshape(1, DH), W_conv, b_conv.reshape(1, DOUT))
    degp = _deg_kernel(ei2, ones1, zeroD)
    g1, ad, b8, di = _factors(degp[:N], degp[NP:NP + N], h)
    sp = _spmm_kernel(g1, ei2, zeroD)
    g2 = _mid(sp[:N], sp[NP:NP + N], h, ad, b8, di)
    sp2 = _spmm_kernel(g2, ei2, zeroD)
    return _final(sp2[:N], sp2[NP:NP + N], h, ad, b8)
